# baseline (speedup 1.0000x reference)
import functools

import jax
import jax.numpy as jnp
from jax import lax
from jax.experimental import pallas as pl
from jax.experimental.pallas import tpu as pltpu
from jax.experimental.pallas import tpu_sc as plsc

B = 16384
D = 64
NEGK = 5
C = 128
LANES = 16


def _sc_scores(pos_u, pos_v, neg_flat, wu2, wv2):
    info = plsc.get_sparse_core_info()
    nw = info.num_cores * info.num_subcores
    epw = B // nw
    nchunk = epw // C
    mesh = plsc.VectorSubcoreMesh(core_axis_name="c", subcore_axis_name="s")

    @functools.partial(
        pl.kernel,
        out_type=[jax.ShapeDtypeStruct((B,), jnp.float32),
                  jax.ShapeDtypeStruct((B * NEGK,), jnp.float32)],
        mesh=mesh,
        scratch_types=[
            pltpu.VMEM((C,), jnp.int32),
            pltpu.VMEM((C,), jnp.int32),
            pltpu.VMEM((C * NEGK,), jnp.int32),
            pltpu.VMEM((C,), jnp.int32),
            pltpu.VMEM((C,), jnp.int32),
            pltpu.VMEM((C * NEGK,), jnp.int32),
            pltpu.VMEM((C, 2 * D), jnp.float32),
            pltpu.VMEM((C, 2 * D), jnp.float32),
            pltpu.VMEM((C * NEGK, 2 * D), jnp.float32),
            pltpu.VMEM((C,), jnp.float32),
            pltpu.VMEM((C * NEGK,), jnp.float32),
            pltpu.SemaphoreType.DMA,
        ],
        compiler_params=pltpu.CompilerParams(needs_layout_passes=False,
                                             use_tc_tiling_on_sc=False),
    )
    def scores(pos_u_hbm, pos_v_hbm, neg_hbm, wu_hbm, wv_hbm,
               pos_out, neg_out, iu, iv, ineg, pru, prv, prn,
               ru, rv, rn, sp, sn, sem):
        wid = lax.axis_index("s") * info.num_cores + lax.axis_index("c")
        lane = jnp.arange(LANES, dtype=jnp.int32)
        for chunk in range(nchunk):
            b0 = wid * epw + chunk * C
            pltpu.sync_copy(pos_u_hbm.at[pl.ds(b0, C)], iu)
            pltpu.sync_copy(pos_v_hbm.at[pl.ds(b0, C)], iv)
            pltpu.sync_copy(neg_hbm.at[pl.ds(b0 * NEGK, C * NEGK)], ineg)
            for g in range(C // LANES):
                s = pl.ds(g * LANES, LANES)
                pru[s] = iu[s] >> 1
                prv[s] = iv[s] >> 1
            for g in range(C * NEGK // LANES):
                s = pl.ds(g * LANES, LANES)
                prn[s] = ineg[s] >> 1
            cp_u = pltpu.async_copy(wu_hbm.at[pru], ru, sem)
            cp_v = pltpu.async_copy(wv_hbm.at[prv], rv, sem)
            cp_n = pltpu.async_copy(wv_hbm.at[prn], rn, sem)
            cp_u.wait()
            cp_v.wait()
            cp_n.wait()
            for g in range(C // LANES):
                s = pl.ds(g * LANES, LANES)
                rowu = lane + (g * LANES)
                rown = [rowu * NEGK + n for n in range(NEGK)]
                colu = (iu[s] & 1) * D
                colv = (iv[s] & 1) * D
                coln = [(plsc.load_gather(ineg, [rown[n]]) & 1) * D
                        for n in range(NEGK)]

                def dbody(d, accs, rowu=rowu, rown=rown,
                          colu=colu, colv=colv, coln=coln):
                    dcol = jnp.full((LANES,), d, jnp.int32)
                    xu = plsc.load_gather(ru, [rowu, colu + dcol])
                    xv = plsc.load_gather(rv, [rowu, colv + dcol])
                    out = [accs[0] + xu * xv]
                    for n in range(NEGK):
                        xn = plsc.load_gather(rn, [rown[n], coln[n] + dcol])
                        out.append(accs[1 + n] + xn * xu)
                    return tuple(out)

                z = jnp.zeros((LANES,), jnp.float32)
                accs = lax.fori_loop(0, D, dbody, (z,) * (1 + NEGK))
                sp[s] = accs[0]
                for n in range(NEGK):
                    plsc.store_scatter(sn, [rown[n]], accs[1 + n])
            pltpu.sync_copy(sp, pos_out.at[pl.ds(b0, C)])
            pltpu.sync_copy(sn, neg_out.at[pl.ds(b0 * NEGK, C * NEGK)])

    return scores(pos_u, pos_v, neg_flat, wu2, wv2)


def _loss(pos_s, neg_s):
    pos2 = pos_s.reshape(B // 128, 128)
    neg2 = neg_s.reshape(B * NEGK // 128, 128)

    def body(p_ref, n_ref, o_ref):
        p = jnp.clip(p_ref[...], -6.0, 6.0)
        n = jnp.clip(n_ref[...], -6.0, 6.0)
        lp = jnp.log1p(jnp.exp(-p))
        ln = jnp.log1p(jnp.exp(n))
        o_ref[0, 0] = (jnp.sum(lp) + jnp.sum(ln)) * (1.0 / B)

    out = pl.pallas_call(
        body,
        out_shape=jax.ShapeDtypeStruct((1, 1), jnp.float32),
        out_specs=pl.BlockSpec(memory_space=pltpu.SMEM),
    )(pos2, neg2)
    return out[0, 0]


def kernel(pos_u, pos_v, neg_v, snd_u_weight, snd_v_weight):
    wu2 = snd_u_weight.reshape(snd_u_weight.shape[0] // 2, 2 * D)
    wv2 = snd_v_weight.reshape(snd_v_weight.shape[0] // 2, 2 * D)
    pos_s, neg_s = _sc_scores(pos_u, pos_v, neg_v.reshape(-1), wu2, wv2)
    return _loss(pos_s, neg_s)


# speedup vs baseline: 1.0001x; 1.0001x over previous
import functools

import jax
import jax.numpy as jnp
from jax import lax
from jax.experimental import pallas as pl
from jax.experimental.pallas import tpu as pltpu
from jax.experimental.pallas import tpu_sc as plsc

B = 16384
D = 64
NEGK = 5
C = 128
LANES = 16


def _sc_scores(pos_u, pos_v, neg_flat, wu2, wv2):
    info = plsc.get_sparse_core_info()
    nw = info.num_cores * info.num_subcores
    epw = B // nw
    nchunk = epw // C
    mesh = plsc.VectorSubcoreMesh(core_axis_name="c", subcore_axis_name="s")

    @functools.partial(
        pl.kernel,
        out_type=[jax.ShapeDtypeStruct((B,), jnp.float32),
                  jax.ShapeDtypeStruct((B * NEGK,), jnp.float32)],
        mesh=mesh,
        scratch_types=[
            pltpu.VMEM((C,), jnp.int32),
            pltpu.VMEM((C,), jnp.int32),
            pltpu.VMEM((C * NEGK,), jnp.int32),
            pltpu.VMEM((C,), jnp.int32),
            pltpu.VMEM((C,), jnp.int32),
            pltpu.VMEM((C * NEGK,), jnp.int32),
            pltpu.VMEM((C, 2 * D), jnp.float32),
            pltpu.VMEM((C, 2 * D), jnp.float32),
            pltpu.VMEM((C * NEGK, 2 * D), jnp.float32),
            pltpu.VMEM((C,), jnp.float32),
            pltpu.VMEM((C * NEGK,), jnp.float32),
            pltpu.SemaphoreType.DMA,
        ],
        compiler_params=pltpu.CompilerParams(needs_layout_passes=False,
                                             use_tc_tiling_on_sc=True),
    )
    def scores(pos_u_hbm, pos_v_hbm, neg_hbm, wu_hbm, wv_hbm,
               pos_out, neg_out, iu, iv, ineg, pru, prv, prn,
               ru, rv, rn, sp, sn, sem):
        wid = lax.axis_index("s") * info.num_cores + lax.axis_index("c")
        lane = jnp.arange(LANES, dtype=jnp.int32)
        for chunk in range(nchunk):
            b0 = wid * epw + chunk * C
            pltpu.sync_copy(pos_u_hbm.at[pl.ds(b0, C)], iu)
            pltpu.sync_copy(pos_v_hbm.at[pl.ds(b0, C)], iv)
            pltpu.sync_copy(neg_hbm.at[pl.ds(b0 * NEGK, C * NEGK)], ineg)
            for g in range(C // LANES):
                s = pl.ds(g * LANES, LANES)
                pru[s] = iu[s] >> 1
                prv[s] = iv[s] >> 1
            for g in range(C * NEGK // LANES):
                s = pl.ds(g * LANES, LANES)
                prn[s] = ineg[s] >> 1
            cp_u = pltpu.async_copy(wu_hbm.at[pru], ru, sem)
            cp_v = pltpu.async_copy(wv_hbm.at[prv], rv, sem)
            cp_n = pltpu.async_copy(wv_hbm.at[prn], rn, sem)
            cp_u.wait()
            cp_v.wait()
            cp_n.wait()
            for g in range(C // LANES):
                s = pl.ds(g * LANES, LANES)
                rowu = lane + (g * LANES)
                rown = [rowu * NEGK + n for n in range(NEGK)]
                colu = (iu[s] & 1) * D
                colv = (iv[s] & 1) * D
                coln = [(plsc.load_gather(ineg, [rown[n]]) & 1) * D
                        for n in range(NEGK)]

                def dbody(d, accs, rowu=rowu, rown=rown,
                          colu=colu, colv=colv, coln=coln):
                    dcol = jnp.full((LANES,), d, jnp.int32)
                    xu = plsc.load_gather(ru, [rowu, colu + dcol])
                    xv = plsc.load_gather(rv, [rowu, colv + dcol])
                    out = [accs[0] + xu * xv]
                    for n in range(NEGK):
                        xn = plsc.load_gather(rn, [rown[n], coln[n] + dcol])
                        out.append(accs[1 + n] + xn * xu)
                    return tuple(out)

                z = jnp.zeros((LANES,), jnp.float32)
                accs = lax.fori_loop(0, D, dbody, (z,) * (1 + NEGK))
                sp[s] = accs[0]
                for n in range(NEGK):
                    plsc.store_scatter(sn, [rown[n]], accs[1 + n])
            pltpu.sync_copy(sp, pos_out.at[pl.ds(b0, C)])
            pltpu.sync_copy(sn, neg_out.at[pl.ds(b0 * NEGK, C * NEGK)])

    return scores(pos_u, pos_v, neg_flat, wu2, wv2)


def _loss(pos_s, neg_s):
    pos2 = pos_s.reshape(B // 128, 128)
    neg2 = neg_s.reshape(B * NEGK // 128, 128)

    def body(p_ref, n_ref, o_ref):
        p = jnp.clip(p_ref[...], -6.0, 6.0)
        n = jnp.clip(n_ref[...], -6.0, 6.0)
        lp = jnp.log1p(jnp.exp(-p))
        ln = jnp.log1p(jnp.exp(n))
        o_ref[0, 0] = (jnp.sum(lp) + jnp.sum(ln)) * (1.0 / B)

    out = pl.pallas_call(
        body,
        out_shape=jax.ShapeDtypeStruct((1, 1), jnp.float32),
        out_specs=pl.BlockSpec(memory_space=pltpu.SMEM),
    )(pos2, neg2)
    return out[0, 0]


def kernel(pos_u, pos_v, neg_v, snd_u_weight, snd_v_weight):
    wu2 = snd_u_weight.reshape(snd_u_weight.shape[0] // 2, 2 * D)
    wv2 = snd_v_weight.reshape(snd_v_weight.shape[0] // 2, 2 * D)
    pos_s, neg_s = _sc_scores(pos_u, pos_v, neg_v.reshape(-1), wu2, wv2)
    return _loss(pos_s, neg_s)
